# baseline (speedup 1.0000x reference)
import functools

import jax
import jax.numpy as jnp
from jax import lax
from jax.experimental import pallas as pl
from jax.experimental.pallas import tpu as pltpu
from jax.experimental.pallas import tpu_sc as plsc

_INTERPRET = False

_TEMP = 1.3
_BETA = 1.5
_K_OUT = 100
_K = 128
_NV = 81


def _dot(a, b):
    return lax.dot_general(a, b, (((1,), (0,)), ((), ())),
                           precision=lax.Precision.HIGHEST,
                           preferred_element_type=jnp.float32)


def _row_max_kernel(logits_ref, obj_ref, unk_ref, m_ref):
    x = logits_ref[...]
    cmask = lax.broadcasted_iota(jnp.int32, x.shape, 3) < _NV
    lmax = jnp.max(jnp.where(cmask, x, -1e30), axis=-1)
    obj = jnp.exp(-_TEMP * obj_ref[...])
    u = jax.nn.sigmoid(unk_ref[...])
    s = jax.nn.sigmoid(lmax)
    maxk = jnp.where(s > 0.2, s, 0.0)
    w = 1.0 - _BETA * u
    kpk = jnp.where((u > maxk) | (w < 0.0), 0.0, (obj * maxk) * w)
    punk = (obj * u) * (1.0 - maxk)
    m_ref[...] = jnp.maximum(punk, kpk)


def _row_max(pred_logits, pred_obj, pred_unk):
    B, N, C = pred_logits.shape
    G, SR = 50, 25
    m = pl.pallas_call(
        _row_max_kernel,
        grid=(G,),
        in_specs=[
            pl.BlockSpec((1, SR, 128, C), lambda i: (i, 0, 0, 0)),
            pl.BlockSpec((1, SR, 128), lambda i: (i, 0, 0)),
            pl.BlockSpec((1, SR, 128), lambda i: (i, 0, 0)),
        ],
        out_specs=pl.BlockSpec((1, SR, 128), lambda i: (i, 0, 0)),
        out_shape=jax.ShapeDtypeStruct((G, SR, 128), jnp.float32),
        interpret=_INTERPRET,
    )(
        pred_logits.reshape(G, SR, 128, C),
        pred_obj.reshape(G, SR, 128),
        pred_unk.reshape(G, SR, 128),
    )
    return m.reshape(B, N)


def _thr_kernel(m_ref, thr_ref):
    bits = lax.bitcast_convert_type(m_ref[...], jnp.int32)
    B = bits.shape[0]

    def body(_, lohi):
        lo, hi = lohi
        mid = lo + ((hi - lo + 1) >> 1)
        cnt = jnp.sum((bits >= mid).astype(jnp.int32), axis=1, keepdims=True)
        ge = cnt >= _K
        return jnp.where(ge, mid, lo), jnp.where(ge, hi, mid - 1)

    lo0 = jnp.zeros((B, 1), jnp.int32)
    hi0 = jnp.full((B, 1), 0x7F800000, jnp.int32)
    lo, _ = lax.fori_loop(0, 31, body, (lo0, hi0))
    thr_ref[...] = jnp.broadcast_to(lo[:, :, None], thr_ref.shape)


def _thresholds(M):
    B, N = M.shape
    return pl.pallas_call(
        _thr_kernel,
        grid=(1,),
        in_specs=[pl.BlockSpec((B, N), lambda i: (0, 0))],
        out_specs=pl.BlockSpec((B, 8, 128), lambda i: (0, 0, 0)),
        out_shape=jax.ShapeDtypeStruct((B, 8, 128), jnp.int32),
        interpret=_INTERPRET,
    )(M)


_NW = 32
_RPW = 20000 // 4


def _sc_body(m_hbm, thr_hbm, lgA_hbm, smA_hbm,
             lgg_hbm, smg_hbm, cnt_hbm, cands_hbm, counts_hbm,
             mv, thrv, candv, csv, cntv, candf, countv,
             idx2a, idx2b, idxs, raw_lg, raw_sm, out_lg, out_sm, sem):
    i32 = jnp.int32
    c = lax.axis_index("c")
    s = lax.axis_index("s")
    w = c * 16 + s
    b = w // 4
    q = w % 4
    pltpu.sync_copy(m_hbm.at[pl.ds(w * _RPW, _RPW)], mv.at[pl.ds(0, _RPW)])
    pltpu.sync_copy(thr_hbm.at[pl.ds(b * 16, 16)], thrv)
    tvec = thrv[...]
    lane = lax.iota(i32, 16)

    def body(i, off):
        v = mv[pl.ds(i * 16, 16)]
        bits = plsc.bitcast(v, i32)
        j = i * 16 + lane
        mask = (bits >= tvec) & (j < _RPW)
        mi = jnp.where(mask, 1, 0).astype(i32)
        ranks = plsc.cumsum(mi)
        dest = jnp.clip(off + ranks - 1, 0, _K - 1)
        plsc.store_scatter(candv, [dest], q * _RPW + j, mask=mask)
        return off + jnp.sum(mi)

    cnt = lax.fori_loop(0, (_RPW + 15) // 16, body, jnp.int32(0))
    countv[...] = jnp.broadcast_to(cnt, (16,)).astype(i32)
    pltpu.sync_copy(candv, cands_hbm.at[pl.ds(w * _K, _K)])
    pltpu.sync_copy(countv, counts_hbm.at[pl.ds(w * 16, 16)])
    plsc.subcore_barrier()

    @pl.when(q == 0)
    def _gather():
        pltpu.sync_copy(cands_hbm.at[pl.ds(w * _K, 4 * _K)], csv)
        pltpu.sync_copy(counts_hbm.at[pl.ds(w * 16, 64)], cntv)
        cnts = [cntv[pl.ds(p * 16, 16)] for p in range(4)]
        off1 = cnts[0]
        off2 = off1 + cnts[1]
        off3 = off2 + cnts[2]
        total = off3 + cnts[3]
        for j in range(8):
            sl = j * 16 + lane
            p = (jnp.where(sl >= off1, 1, 0) + jnp.where(sl >= off2, 1, 0)
                 + jnp.where(sl >= off3, 1, 0)).astype(i32)
            offp = (jnp.where(p >= 1, cnts[0], 0)
                    + jnp.where(p >= 2, cnts[1], 0)
                    + jnp.where(p >= 3, cnts[2], 0)).astype(i32)
            col = jnp.clip(sl - offp, 0, _K - 1) + p * _K
            vmask = sl < total
            n = plsc.load_gather(csv, [col], mask=vmask)
            n = jnp.where(vmask, n, 0)
            candf[pl.ds(j * 16, 16)] = n + b * 20000
        for j in range(8):
            sl = j * 16 + lane
            nk = plsc.load_gather(candf, [sl])
            base = (nk * 91) >> 7
            idx2a[pl.ds(j * 16, 16)] = base
            idx2b[pl.ds(j * 16, 16)] = jnp.minimum(base + 1, 113749)
        for j in range(8):
            sl = j * 16 + lane
            nk = plsc.load_gather(candf, [sl])
            idxs[pl.ds(j * 16, 16)] = nk >> 4
        pltpu.async_copy(lgA_hbm.at[idx2a], raw_lg.at[pl.ds(0, _K)], sem).wait()
        pltpu.async_copy(lgA_hbm.at[idx2b], raw_lg.at[pl.ds(_K, _K)], sem).wait()
        pltpu.async_copy(smA_hbm.at[idxs], raw_sm, sem).wait()

        def ext_lg(v, _):
            t = v * 16 + lane
            k = t // 96
            cc = t % 96
            nk = plsc.load_gather(candf, [k])
            pos = (nk * 91 & 127) + cc
            val = plsc.load_gather(raw_lg, [k + _K * (pos >> 7), pos & 127])
            out_lg[pl.ds(v * 16, 16)] = val
            return 0

        lax.fori_loop(0, _K * 96 // 16, ext_lg, 0)

        def ext_sm(v, _):
            t = v * 16 + lane
            k = t >> 3
            col = t & 7
            nk = plsc.load_gather(candf, [k])
            val = plsc.load_gather(raw_sm, [k, (nk & 15) * 8 + col])
            out_sm[pl.ds(v * 16, 16)] = val
            return 0

        lax.fori_loop(0, _K * 8 // 16, ext_sm, 0)
        pltpu.sync_copy(out_lg, lgg_hbm.at[pl.ds(b * _K * 96, _K * 96)])
        pltpu.sync_copy(out_sm, smg_hbm.at[pl.ds(b * _K * 8, _K * 8)])
        countv[...] = jnp.minimum(total, _K)
        pltpu.sync_copy(countv, cnt_hbm.at[pl.ds(b * 16, 16)])


def _sc_compact_gather(m1d, thrv, logits2, small2):
    mesh = plsc.VectorSubcoreMesh(core_axis_name="c", subcore_axis_name="s",
                                  num_cores=2, num_subcores=16)
    f32, i32 = jnp.float32, jnp.int32
    k = functools.partial(
        pl.kernel,
        compiler_params=pltpu.CompilerParams(needs_layout_passes=False),
        out_type=[
            jax.ShapeDtypeStruct((8 * _K * 96,), f32),
            jax.ShapeDtypeStruct((8 * _K * 8,), f32),
            jax.ShapeDtypeStruct((8 * 16,), i32),
            jax.ShapeDtypeStruct((_NW * _K,), i32),
            jax.ShapeDtypeStruct((_NW * 16,), i32),
        ],
        mesh=mesh,
        scratch_types=[
            pltpu.VMEM((_RPW + 16,), f32),
            pltpu.VMEM((16,), i32),
            pltpu.VMEM((_K,), i32),
            pltpu.VMEM((4 * _K,), i32),
            pltpu.VMEM((64,), i32),
            pltpu.VMEM((_K,), i32),
            pltpu.VMEM((16,), i32),
            pltpu.VMEM((_K,), i32),
            pltpu.VMEM((_K,), i32),
            pltpu.VMEM((_K,), i32),
            pltpu.VMEM((2 * _K, 128), f32),
            pltpu.VMEM((_K, 128), f32),
            pltpu.VMEM((_K * 96,), f32),
            pltpu.VMEM((_K * 8,), f32),
            pltpu.SemaphoreType.DMA,
        ],
    )(_sc_body)
    lgg, smg, cnts, _, _ = k(m1d, thrv, logits2, small2)
    return lgg.reshape(8, _K, 96), smg.reshape(8, _K, 8), cnts


def _select_kernel(lg_ref, sm_ref, cnt_ref, tsz_ref,
                   sc_ref, lb_ref, bx_ref):
    C = lg_ref.shape[2]
    K = _K
    f32, i32 = jnp.float32, jnp.int32

    LG = lg_ref[0]
    SM = sm_ref[0]
    cnt1 = jnp.max(cnt_ref[0])
    ir = lax.broadcasted_iota(i32, (K, K), 0)
    ic = lax.broadcasted_iota(i32, (K, K), 1)
    eye = (ic == ir).astype(f32)
    valid = lax.broadcasted_iota(i32, (K, 1), 0) < cnt1

    obj = jnp.exp(-_TEMP * SM[:, 0:1])
    u = jax.nn.sigmoid(SM[:, 1:2])
    kp = jax.nn.sigmoid(LG)
    kp = kp * (kp > 0.2).astype(f32)
    cl = lax.broadcasted_iota(i32, (K, C), 1)
    kp = jnp.where(cl >= _NV, 0.0, kp)
    maxk = jnp.max(jnp.where(cl < 90, kp, 0.0), axis=1, keepdims=True)
    w = 1.0 - _BETA * u
    pk = jnp.where(u > maxk, 0.0, (obj * kp) * w)
    pu = (obj * u) * (1.0 - maxk)
    ent = jnp.where(cl == 90, pu, pk)
    ent = jnp.where(valid, ent, -1.0)

    ebits = lax.bitcast_convert_type(ent, i32)

    def body(_, lohi):
        lo, hi = lohi
        mid = lo + ((hi - lo + 1) >> 1)
        cnt = jnp.sum((ebits >= mid).astype(i32))
        ge = cnt >= _K_OUT
        return jnp.where(ge, mid, lo), jnp.where(ge, hi, mid - 1)

    lo2, _ = lax.fori_loop(
        0, 31, body, (jnp.int32(0), jnp.int32(0x7F800000)))

    q2 = (ebits >= lo2).astype(f32)
    rowsum = jnp.sum(q2, axis=1, keepdims=True)
    Lt = (ic < ir).astype(f32)
    ones_row = jnp.ones((1, K), f32)
    rowoff = _dot(Lt, rowsum)
    rowoffT = _dot(ones_row, eye * rowoff)
    rowendT = _dot(ones_row, eye * (rowoff + rowsum))
    sidx = lax.broadcasted_iota(i32, (K, K), 0).astype(f32)
    R2 = ((rowoffT <= sidx) & (sidx < rowendT)).astype(f32)
    RV = _dot(R2, ent)
    QR = _dot(R2, q2)
    SMg = _dot(R2, SM)
    s_local = lax.broadcasted_iota(i32, (K, 1), 0).astype(f32) - _dot(R2, rowoff)
    cumrow = QR
    sh = 1
    while sh < C:
        cumrow = cumrow + jnp.concatenate(
            [jnp.zeros((K, sh), f32), cumrow[:, : C - sh]], axis=1)
        sh *= 2
    C1 = ((cumrow == s_local + 1.0) & (QR > 0.5)).astype(f32)
    val = jnp.sum(C1 * RV, axis=1, keepdims=True)
    clf = lax.broadcasted_iota(i32, (K, C), 1).astype(f32)
    lab = jnp.sum(C1 * clf, axis=1, keepdims=True)

    valT = _dot(ones_row, eye * val)
    beats = (valT > val) | ((valT == val) & (ic < ir))
    rank = jnp.sum(beats.astype(f32), axis=1, keepdims=True)
    rankT = _dot(ones_row, eye * rank)
    F = (rankT == sidx).astype(f32)
    sc = _dot(F, val)
    lb = _dot(F, lab)
    bxg = _dot(F, SMg[:, 2:6])

    cx, cy, bw, bh = bxg[:, 0:1], bxg[:, 1:2], bxg[:, 2:3], bxg[:, 3:4]
    h_img = tsz_ref[0, :, 0:1]
    w_img = tsz_ref[0, :, 1:2]
    x0 = (cx - 0.5 * bw) * w_img
    y0 = (cy - 0.5 * bh) * h_img
    x1 = (cx + 0.5 * bw) * w_img
    y1 = (cy + 0.5 * bh) * h_img
    sc_ref[...] = sc[None]
    lb_ref[...] = lb.astype(i32)[None]
    bx_ref[...] = jnp.concatenate([x0, y0, x1, y1], axis=1)[None]


def _select(lgg, smg, cnts, tszf):
    B, C = lgg.shape[0], lgg.shape[2]
    return pl.pallas_call(
        _select_kernel,
        grid=(B,),
        in_specs=[
            pl.BlockSpec((1, _K, 96), lambda b: (b, 0, 0)),
            pl.BlockSpec((1, _K, 8), lambda b: (b, 0, 0)),
            pl.BlockSpec((1, 1, 16), lambda b: (b, 0, 0)),
            pl.BlockSpec((1, 1, 2), lambda b: (b, 0, 0)),
        ],
        out_specs=[
            pl.BlockSpec((1, _K, 1), lambda b: (b, 0, 0)),
            pl.BlockSpec((1, _K, 1), lambda b: (b, 0, 0)),
            pl.BlockSpec((1, _K, 4), lambda b: (b, 0, 0)),
        ],
        out_shape=[
            jax.ShapeDtypeStruct((B, _K, 1), jnp.float32),
            jax.ShapeDtypeStruct((B, _K, 1), jnp.int32),
            jax.ShapeDtypeStruct((B, _K, 4), jnp.float32),
        ],
        interpret=_INTERPRET,
    )(lgg, smg, cnts.reshape(B, 1, 16), tszf)


def kernel(pred_logits, pred_obj, pred_boxes, pred_unk, target_sizes):
    B, N, C = pred_logits.shape
    M = _row_max(pred_logits, pred_obj, pred_unk)
    thr = _thresholds(M)
    small = jnp.concatenate(
        [pred_obj[..., None], pred_unk[..., None], pred_boxes,
         jnp.zeros((B, N, 2), jnp.float32)], axis=-1)
    lgg, smg, cnts = _sc_compact_gather(
        M.reshape(B * N), thr[:, 0, :16].reshape(B * 16),
        pred_logits.reshape(B * N * C // 128, 128),
        small.reshape(B * N * 8 // 128, 128))
    tszf = target_sizes.astype(jnp.float32).reshape(B, 1, 2)
    sc, lb, bx = _select(lgg, smg, cnts, tszf)
    return (sc[:, :_K_OUT, 0], lb[:, :_K_OUT, 0], bx[:, :_K_OUT, :])
